# baseline (speedup 1.0000x reference)
import functools

import jax
import jax.numpy as jnp
from jax import lax
from jax.experimental import pallas as pl
from jax.experimental.pallas import tpu as pltpu

_BASE_ALPHA = 0.05
_NUM_ITER = 2
_CHUNK = 512


def _refine_body(pred_ref, partial_ref, out_ref, mind2_ref, idx_ref):
    n = pred_ref.shape[1]
    m = partial_ref.shape[1]
    d = pred_ref.shape[2]
    n_chunks = n // _CHUNK

    part = partial_ref[0]
    part_bf = part.astype(jnp.bfloat16)
    k2 = jnp.sum(part * part, axis=1)[None, :]
    iota_m = lax.broadcasted_iota(jnp.int32, (_CHUNK, m), 1).astype(jnp.float32)
    iota_row = lax.broadcasted_iota(jnp.int32, (1, m), 1).astype(jnp.float32)

    for it in range(_NUM_ITER):
        src_ref = pred_ref if it == 0 else out_ref

        def pass1(c, running_max):
            q = src_ref[0, pl.ds(c * _CHUNK, _CHUNK), :]
            qk = lax.dot_general(-2.0 * q, part, (((1,), (1,)), ((), ())),
                                 preferred_element_type=jnp.float32)
            s = qk + k2
            mn = jnp.min(s, axis=1, keepdims=True)
            idx = jnp.argmin(s, axis=1).astype(jnp.float32)[:, None]
            q2 = jnp.sum(q * q, axis=1, keepdims=True)
            mind2_ref[c] = q2 + mn
            idx_ref[c] = idx
            return jnp.maximum(running_max, jnp.max(q2 + mn))

        max_d2 = lax.fori_loop(0, n_chunks, pass1, jnp.float32(-jnp.inf))
        denom = jnp.sqrt(jnp.maximum(max_d2, 1e-12)) + 1e-6

        def pass2(c, _):
            idx = idx_ref[c]
            onehot = jnp.where(idx == iota_row, 1.0, 0.0
                               ).astype(jnp.bfloat16)
            nearest = lax.dot_general(onehot, part_bf,
                                      (((1,), (0,)), ((), ())),
                                      preferred_element_type=jnp.float32)
            mind = jnp.sqrt(jnp.maximum(mind2_ref[c], 1e-12))
            alpha = _BASE_ALPHA * (2.0 - mind / denom)
            q = src_ref[0, pl.ds(c * _CHUNK, _CHUNK), :]
            out_ref[0, pl.ds(c * _CHUNK, _CHUNK), :] = (
                q + alpha * (nearest - q))
            return 0

        lax.fori_loop(0, n_chunks, pass2, 0)


@jax.jit
def kernel(pred, partial):
    b, n, d = pred.shape
    _, m, _ = partial.shape
    n_chunks = n // _CHUNK
    return pl.pallas_call(
        _refine_body,
        grid=(b,),
        in_specs=[
            pl.BlockSpec((1, n, d), lambda i: (i, 0, 0)),
            pl.BlockSpec((1, m, d), lambda i: (i, 0, 0)),
        ],
        out_specs=pl.BlockSpec((1, n, d), lambda i: (i, 0, 0)),
        out_shape=jax.ShapeDtypeStruct((b, n, d), jnp.float32),
        scratch_shapes=[
            pltpu.VMEM((n_chunks, _CHUNK, 1), jnp.float32),
            pltpu.VMEM((n_chunks, _CHUNK, 1), jnp.float32),
        ],
        compiler_params=pltpu.CompilerParams(
            dimension_semantics=("arbitrary",),
        ),
    )(pred, partial)


# speedup vs baseline: 1.3930x; 1.3930x over previous
import functools

import jax
import jax.numpy as jnp
from jax import lax
from jax.experimental import pallas as pl
from jax.experimental.pallas import tpu as pltpu

_BASE_ALPHA = 0.05
_NUM_ITER = 2
_CHUNK = 512


def _refine_body(pred_ref, partial_ref, out_ref, mind2_ref, idx_ref):
    n = pred_ref.shape[1]
    m = partial_ref.shape[1]
    d = pred_ref.shape[2]
    n_chunks = n // _CHUNK

    part = partial_ref[0]
    part_bf = part.astype(jnp.bfloat16)
    k2 = jnp.sum(part * part, axis=1)[None, :]
    iota_m = lax.broadcasted_iota(jnp.int32, (_CHUNK, m), 1).astype(jnp.float32)
    iota_row = lax.broadcasted_iota(jnp.int32, (1, m), 1).astype(jnp.float32)

    for it in range(_NUM_ITER):
        src_ref = pred_ref if it == 0 else out_ref

        def pass1(c, running_max):
            q = src_ref[0, pl.ds(c * _CHUNK, _CHUNK), :]
            qk = lax.dot_general(-2.0 * q, part, (((1,), (1,)), ((), ())),
                                 preferred_element_type=jnp.float32)
            s = qk + k2
            mn = jnp.min(s, axis=1, keepdims=True)
            idx = jnp.min(jnp.where(s <= mn, iota_m, float(m)), axis=1,
                          keepdims=True)
            q2 = jnp.sum(q * q, axis=1, keepdims=True)
            mind2_ref[c] = q2 + mn
            idx_ref[c] = idx
            return jnp.maximum(running_max, jnp.max(q2 + mn))

        max_d2 = jnp.float32(-jnp.inf)
        for c in range(n_chunks):
            max_d2 = pass1(c, max_d2)
        denom = jnp.sqrt(jnp.maximum(max_d2, 1e-12)) + 1e-6

        def pass2(c, _):
            idx = idx_ref[c]
            onehot = jnp.where(idx == iota_row, 1.0, 0.0
                               ).astype(jnp.bfloat16)
            nearest = lax.dot_general(onehot, part_bf,
                                      (((1,), (0,)), ((), ())),
                                      preferred_element_type=jnp.float32)
            mind = jnp.sqrt(jnp.maximum(mind2_ref[c], 1e-12))
            alpha = _BASE_ALPHA * (2.0 - mind / denom)
            q = src_ref[0, pl.ds(c * _CHUNK, _CHUNK), :]
            out_ref[0, pl.ds(c * _CHUNK, _CHUNK), :] = (
                q + alpha * (nearest - q))
            return 0

        for c in range(n_chunks):
            pass2(c, 0)


@jax.jit
def kernel(pred, partial):
    b, n, d = pred.shape
    _, m, _ = partial.shape
    n_chunks = n // _CHUNK
    return pl.pallas_call(
        _refine_body,
        grid=(b,),
        in_specs=[
            pl.BlockSpec((1, n, d), lambda i: (i, 0, 0)),
            pl.BlockSpec((1, m, d), lambda i: (i, 0, 0)),
        ],
        out_specs=pl.BlockSpec((1, n, d), lambda i: (i, 0, 0)),
        out_shape=jax.ShapeDtypeStruct((b, n, d), jnp.float32),
        scratch_shapes=[
            pltpu.VMEM((n_chunks, _CHUNK, 1), jnp.float32),
            pltpu.VMEM((n_chunks, _CHUNK, 1), jnp.float32),
        ],
        compiler_params=pltpu.CompilerParams(
            dimension_semantics=("arbitrary",),
        ),
    )(pred, partial)


# speedup vs baseline: 1.6037x; 1.1512x over previous
import functools

import jax
import jax.numpy as jnp
from jax import lax
from jax.experimental import pallas as pl
from jax.experimental.pallas import tpu as pltpu

_BASE_ALPHA = 0.05
_NUM_ITER = 2
_CHUNK = 512


def _refine_body(pred_ref, partial_ref, out_ref, mind2_ref, mask_ref):
    n = pred_ref.shape[1]
    m = partial_ref.shape[1]
    d = pred_ref.shape[2]
    n_chunks = n // _CHUNK

    part = partial_ref[0]
    part1_bf = jnp.concatenate(
        [part, jnp.ones((m, 1), jnp.float32)], axis=1
    ).astype(jnp.bfloat16)
    k2 = jnp.sum(part * part, axis=1)[None, :]

    for it in range(_NUM_ITER):
        src_ref = pred_ref if it == 0 else out_ref

        def pass1(c, running_max):
            q = src_ref[0, pl.ds(c * _CHUNK, _CHUNK), :]
            qk = lax.dot_general(-2.0 * q, part, (((1,), (1,)), ((), ())),
                                 preferred_element_type=jnp.float32)
            s = qk + k2
            mn = jnp.min(s, axis=1, keepdims=True)
            mask_ref[c] = jnp.where(s <= mn, 1.0, 0.0
                                    ).astype(jnp.bfloat16)
            q2 = jnp.sum(q * q, axis=1, keepdims=True)
            mind2_ref[c] = q2 + mn
            return jnp.maximum(running_max, jnp.max(q2 + mn))

        max_d2 = jnp.float32(-jnp.inf)
        for c in range(n_chunks):
            max_d2 = pass1(c, max_d2)
        denom = jnp.sqrt(jnp.maximum(max_d2, 1e-12)) + 1e-6

        def pass2(c):
            g = lax.dot_general(mask_ref[c], part1_bf,
                                (((1,), (0,)), ((), ())),
                                preferred_element_type=jnp.float32)
            nearest = g[:, :d] / g[:, d:]
            mind = jnp.sqrt(jnp.maximum(mind2_ref[c], 1e-12))
            alpha = _BASE_ALPHA * (2.0 - mind / denom)
            q = src_ref[0, pl.ds(c * _CHUNK, _CHUNK), :]
            out_ref[0, pl.ds(c * _CHUNK, _CHUNK), :] = (
                q + alpha * (nearest - q))

        for c in range(n_chunks):
            pass2(c)


@jax.jit
def kernel(pred, partial):
    b, n, d = pred.shape
    _, m, _ = partial.shape
    n_chunks = n // _CHUNK
    return pl.pallas_call(
        _refine_body,
        grid=(b,),
        in_specs=[
            pl.BlockSpec((1, n, d), lambda i: (i, 0, 0)),
            pl.BlockSpec((1, m, d), lambda i: (i, 0, 0)),
        ],
        out_specs=pl.BlockSpec((1, n, d), lambda i: (i, 0, 0)),
        out_shape=jax.ShapeDtypeStruct((b, n, d), jnp.float32),
        scratch_shapes=[
            pltpu.VMEM((n_chunks, _CHUNK, 1), jnp.float32),
            pltpu.VMEM((n_chunks, _CHUNK, m), jnp.bfloat16),
        ],
        compiler_params=pltpu.CompilerParams(
            dimension_semantics=("arbitrary",),
        ),
    )(pred, partial)
